# baseline (speedup 1.0000x reference)
import jax
import jax.numpy as jnp
from jax import lax
from jax.experimental import pallas as pl
from jax.experimental.pallas import tpu as pltpu
from jax.experimental.pallas import tpu_sc as plsc

VOCAB = 1000000
BATCH = 16384
TOKENS = 20
D = 64
NC, NS = 2, 16
NW = NC * NS
DOCS_PER_W = BATCH // NW
CHUNK_DOCS = 32
ROWS_PER_CHUNK = CHUNK_DOCS * TOKENS
GATHERS_PER_CHUNK = ROWS_PER_CHUNK // 128
CHUNKS = DOCS_PER_W // CHUNK_DOCS

PACK_BLK = 5000
NPACK = VOCAB // 4


def _round_bf16_bits(y):
    return lax.bitcast_convert_type(y, jnp.int32) + 0x8000


def _pack_kernel(t0_ref, t1_ref, t2_ref, t3_ref, w_ref, o_ref):
    for q, t_ref in enumerate((t0_ref, t1_ref, t2_ref, t3_ref)):
        y = lax.dot_general(
            t_ref[...], w_ref[...], (((1,), (1,)), ((), ())),
            preferred_element_type=jnp.float32,
        )
        r = _round_bf16_bits(y)
        ra = r[:, :32]
        rb = r[:, 32:]
        w32 = (lax.shift_right_logical(ra, 16) | (rb & jnp.int32(-65536)))
        o_ref[:, pl.ds(32 * q, 32)] = w32


def _pack_tc(table, W):
    steps = NPACK // PACK_BLK

    def t_spec(q):
        return pl.BlockSpec((PACK_BLK, D), lambda i, q=q: (i + q * steps, 0))

    return pl.pallas_call(
        _pack_kernel,
        out_shape=jax.ShapeDtypeStruct((NPACK, 128), jnp.int32),
        grid=(steps,),
        in_specs=[
            t_spec(0), t_spec(1), t_spec(2), t_spec(3),
            pl.BlockSpec((D, D), lambda i: (0, 0)),
        ],
        out_specs=pl.BlockSpec((PACK_BLK, 128), lambda i: (i, 0)),
    )(table, table, table, table, W)


def _pool_sc_kernel(idx_hbm, off_hbm, packed_hbm, out_hbm,
                    idx_c, off_c, rows_v, out_v, sem):
    wid = lax.axis_index("s") * NC + lax.axis_index("c")

    @pl.loop(0, CHUNKS)
    def _chunk(c):
        g = wid * CHUNKS + c
        doc_base = g * CHUNK_DOCS
        pltpu.sync_copy(idx_hbm.at[g], idx_c)
        pltpu.sync_copy(off_hbm.at[g], off_c)
        descs = []
        for r in range(GATHERS_PER_CHUNK):
            descs.append(
                pltpu.async_copy(
                    packed_hbm.at[idx_c.at[r]],
                    rows_v.at[pl.ds(128 * r, 128)],
                    sem,
                )
            )
        for desc in descs:
            desc.wait()

        @pl.loop(0, CHUNK_DOCS)
        def _doc(d):
            row0 = d * TOKENS
            v0 = off_c[2 * d, pl.ds(0, 16)]
            v1 = off_c[2 * d + 1, pl.ds(0, 16)]
            zero = jnp.zeros((16,), jnp.float32)
            accs = [zero, zero, zero, zero]
            for t in range(TOKENS):
                o = v0[t] if t < 16 else v1[t - 16]
                x0 = rows_v[row0 + t, pl.ds(o, 16)]
                x1 = rows_v[row0 + t, pl.ds(o + 16, 16)]
                accs[0] = accs[0] + lax.bitcast_convert_type(
                    lax.shift_left(x0, 16), jnp.float32)
                accs[2] = accs[2] + lax.bitcast_convert_type(
                    x0 & jnp.int32(-65536), jnp.float32)
                accs[1] = accs[1] + lax.bitcast_convert_type(
                    lax.shift_left(x1, 16), jnp.float32)
                accs[3] = accs[3] + lax.bitcast_convert_type(
                    x1 & jnp.int32(-65536), jnp.float32)
            for k in range(4):
                out_v[d, pl.ds(k * 16, 16)] = accs[k]

        pltpu.sync_copy(out_v, out_hbm.at[pl.ds(doc_base, CHUNK_DOCS)])


def _pool_sc(idx, off, packed):
    mesh = plsc.VectorSubcoreMesh(
        core_axis_name="c", subcore_axis_name="s", num_cores=NC, num_subcores=NS
    )
    f = pl.kernel(
        _pool_sc_kernel,
        out_type=jax.ShapeDtypeStruct((BATCH, D), jnp.float32),
        mesh=mesh,
        scratch_types=[
            pltpu.VMEM((GATHERS_PER_CHUNK, 128), jnp.int32),
            pltpu.VMEM((2 * CHUNK_DOCS, 16), jnp.int32),
            pltpu.VMEM((ROWS_PER_CHUNK, 128), jnp.int32),
            pltpu.VMEM((CHUNK_DOCS, D), jnp.float32),
            pltpu.SemaphoreType.DMA,
        ],
    )
    return f(idx, off, packed)


def kernel(document, table, W):
    packed = _pack_tc(table, W)
    tok = document[:, :TOKENS]
    idx = (tok % NPACK).reshape(NW * CHUNKS, GATHERS_PER_CHUNK, 128)
    off = jnp.pad((tok // NPACK) * 32, ((0, 0), (0, 32 - TOKENS)))
    off = off.reshape(NW * CHUNKS, 2 * CHUNK_DOCS, 16)
    return _pool_sc(idx, off, packed)


# speedup vs baseline: 1.0477x; 1.0477x over previous
import jax
import jax.numpy as jnp
from jax import lax
from jax.experimental import pallas as pl
from jax.experimental.pallas import tpu as pltpu
from jax.experimental.pallas import tpu_sc as plsc

VOCAB = 1000000
BATCH = 16384
TOKENS = 20
D = 64
NC, NS = 2, 16
NW = NC * NS
DOCS_PER_W = BATCH // NW
CHUNK_DOCS = 32
ROWS_PER_CHUNK = CHUNK_DOCS * TOKENS
GATHERS_PER_CHUNK = ROWS_PER_CHUNK // 128
CHUNKS = DOCS_PER_W // CHUNK_DOCS

PACK_BLK = 10000
NPACK = VOCAB // 4


def _round_bf16_bits(y):
    return lax.bitcast_convert_type(y, jnp.int32) + 0x8000


def _pack_kernel(t0_ref, t1_ref, t2_ref, t3_ref, w_ref, o_ref):
    for q, t_ref in enumerate((t0_ref, t1_ref, t2_ref, t3_ref)):
        y = lax.dot_general(
            t_ref[...], w_ref[...], (((1,), (1,)), ((), ())),
            preferred_element_type=jnp.float32,
        )
        r = _round_bf16_bits(y)
        ra = r[:, :32]
        rb = r[:, 32:]
        w32 = (lax.shift_right_logical(ra, 16) | (rb & jnp.int32(-65536)))
        o_ref[:, pl.ds(32 * q, 32)] = w32


def _pack_tc(table, W):
    steps = NPACK // PACK_BLK

    def t_spec(q):
        return pl.BlockSpec((PACK_BLK, D), lambda i, q=q: (i + q * steps, 0))

    return pl.pallas_call(
        _pack_kernel,
        out_shape=jax.ShapeDtypeStruct((NPACK, 128), jnp.int32),
        grid=(steps,),
        in_specs=[
            t_spec(0), t_spec(1), t_spec(2), t_spec(3),
            pl.BlockSpec((D, D), lambda i: (0, 0)),
        ],
        out_specs=pl.BlockSpec((PACK_BLK, 128), lambda i: (i, 0)),
    )(table, table, table, table, W)


def _pool_sc_kernel(idx_hbm, off_hbm, packed_hbm, out_hbm,
                    idx_c, off_c, rows_v, out_v, sem):
    wid = lax.axis_index("s") * NC + lax.axis_index("c")

    @pl.loop(0, CHUNKS)
    def _chunk(c):
        g = wid * CHUNKS + c
        doc_base = g * CHUNK_DOCS
        pltpu.sync_copy(idx_hbm.at[g], idx_c)
        pltpu.sync_copy(off_hbm.at[g], off_c)
        descs = []
        for r in range(GATHERS_PER_CHUNK):
            descs.append(
                pltpu.async_copy(
                    packed_hbm.at[idx_c.at[r]],
                    rows_v.at[pl.ds(128 * r, 128)],
                    sem,
                )
            )
        for desc in descs:
            desc.wait()

        @pl.loop(0, CHUNK_DOCS)
        def _doc(d):
            row0 = d * TOKENS
            v0 = off_c[2 * d, pl.ds(0, 16)]
            v1 = off_c[2 * d + 1, pl.ds(0, 16)]
            zero = jnp.zeros((16,), jnp.float32)
            accs = [zero, zero, zero, zero]
            for t in range(TOKENS):
                o = v0[t] if t < 16 else v1[t - 16]
                x0 = rows_v[row0 + t, pl.ds(o, 16)]
                x1 = rows_v[row0 + t, pl.ds(o + 16, 16)]
                accs[0] = accs[0] + lax.bitcast_convert_type(
                    lax.shift_left(x0, 16), jnp.float32)
                accs[2] = accs[2] + lax.bitcast_convert_type(
                    x0 & jnp.int32(-65536), jnp.float32)
                accs[1] = accs[1] + lax.bitcast_convert_type(
                    lax.shift_left(x1, 16), jnp.float32)
                accs[3] = accs[3] + lax.bitcast_convert_type(
                    x1 & jnp.int32(-65536), jnp.float32)
            for k in range(4):
                out_v[d, pl.ds(k * 16, 16)] = accs[k]

        pltpu.sync_copy(out_v, out_hbm.at[pl.ds(doc_base, CHUNK_DOCS)])


def _pool_sc(idx, off, packed):
    mesh = plsc.VectorSubcoreMesh(
        core_axis_name="c", subcore_axis_name="s", num_cores=NC, num_subcores=NS
    )
    f = pl.kernel(
        _pool_sc_kernel,
        out_type=jax.ShapeDtypeStruct((BATCH, D), jnp.float32),
        mesh=mesh,
        scratch_types=[
            pltpu.VMEM((GATHERS_PER_CHUNK, 128), jnp.int32),
            pltpu.VMEM((2 * CHUNK_DOCS, 16), jnp.int32),
            pltpu.VMEM((ROWS_PER_CHUNK, 128), jnp.int32),
            pltpu.VMEM((CHUNK_DOCS, D), jnp.float32),
            pltpu.SemaphoreType.DMA,
        ],
    )
    return f(idx, off, packed)


def kernel(document, table, W):
    packed = _pack_tc(table, W)
    tok = document[:, :TOKENS]
    idx = (tok % NPACK).reshape(NW * CHUNKS, GATHERS_PER_CHUNK, 128)
    off = jnp.pad((tok // NPACK) * 32, ((0, 0), (0, 32 - TOKENS)))
    off = off.reshape(NW * CHUNKS, 2 * CHUNK_DOCS, 16)
    return _pool_sc(idx, off, packed)
